# baseline (speedup 1.0000x reference)
import functools

import jax
import jax.numpy as jnp
from jax import lax
from jax.experimental import pallas as pl
from jax.experimental.pallas import tpu as pltpu
from jax.experimental.pallas import tpu_sc as plsc

S, H, E, DFF = 4096, 1024, 8, 4096
T = 256
NF = 4
FB = DFF // NF
NT = 40
NPAD = NT * T
EPAD = 128

NC, NS = 2, 16
NW = NC * NS
TPW = S // NW
CH = 64


def _shift_down(a, k):
    z = jnp.zeros((k,) + a.shape[1:], a.dtype)
    return jnp.concatenate([z, a[:-k]], axis=0)


def _router_body(x_ref, wr_ref, br_ref,
                 pos0_ref, pos1_ref, w0_ref, w1_ref, te_ref, tv_ref, bal_ref):
    xv = x_ref[...]
    scores_p = jnp.dot(xv, wr_ref[...], preferred_element_type=jnp.float32)
    scores = scores_p[:, 0:E] + br_ref[...]
    m = jnp.max(scores, axis=1, keepdims=True)
    ex = jnp.exp(scores - m)
    probs = ex / jnp.sum(ex, axis=1, keepdims=True)

    iota_e = jax.lax.broadcasted_iota(jnp.int32, (S, E), 1)
    p0 = jnp.max(probs, axis=1, keepdims=True)
    i0 = jnp.min(jnp.where(probs == p0, iota_e, E), axis=1, keepdims=True)
    oh0 = (iota_e == i0).astype(jnp.int32)
    probs2 = jnp.where(oh0 == 1, -jnp.inf, probs)
    p1 = jnp.max(probs2, axis=1, keepdims=True)
    i1 = jnp.min(jnp.where(probs2 == p1, iota_e, E), axis=1, keepdims=True)
    oh1 = (iota_e == i1).astype(jnp.int32)

    psum = p0 + p1
    w0_ref[...] = p0 / psum
    w1_ref[...] = p1 / psum

    cb = jnp.concatenate([oh0, oh1], axis=1)
    k = 1
    while k < S:
        cb = cb + _shift_down(cb, k)
        k *= 2
    c0 = cb[:, 0:E]
    c1 = cb[:, E:2 * E]
    total0 = c0[S - 1:S, :]
    total1 = c1[S - 1:S, :]
    counts = total0 + total1

    cf = counts.astype(jnp.float32)
    ntT = jnp.ceil(cf / T) * T
    cum = ntT
    for kk in (1, 2, 4):
        z = jnp.zeros((1, kk), cum.dtype)
        cum = cum + jnp.concatenate([z, cum[:, :-kk]], axis=1)
    pstart = (cum - ntT).astype(jnp.int32)

    rank0 = jnp.sum((c0 - 1) * oh0, axis=1, keepdims=True)
    rank1 = jnp.sum((c1 - 1) * oh1 + total0 * oh1, axis=1, keepdims=True)
    pos0_ref[...] = rank0 + jnp.sum(pstart * oh0, axis=1, keepdims=True)
    pos1_ref[...] = rank1 + jnp.sum(pstart * oh1, axis=1, keepdims=True)

    cum_t = (cum * (1.0 / T)).astype(jnp.int32)
    t_iota = jax.lax.broadcasted_iota(jnp.int32, (128, E), 0)
    te = jnp.sum((t_iota >= cum_t).astype(jnp.int32), axis=1, keepdims=True)
    te_ref[...] = jnp.minimum(te, E - 1)
    tv_ref[...] = (t_iota[:, 0:1] < cum_t[:, E - 1:E]).astype(jnp.int32)

    avgp = jnp.sum(probs, axis=0, keepdims=True) * (1.0 / S)
    freqs = total0.astype(jnp.float32) * (1.0 / S)
    bal_ref[...] = 0.001 * jnp.sum(freqs * avgp, axis=1, keepdims=True)


def _router(x2, wr_p, br_row):
    out_shapes = (
        jax.ShapeDtypeStruct((S, 1), jnp.int32),
        jax.ShapeDtypeStruct((S, 1), jnp.int32),
        jax.ShapeDtypeStruct((S, 1), jnp.float32),
        jax.ShapeDtypeStruct((S, 1), jnp.float32),
        jax.ShapeDtypeStruct((128, 1), jnp.int32),
        jax.ShapeDtypeStruct((128, 1), jnp.int32),
        jax.ShapeDtypeStruct((1, 1), jnp.float32),
    )
    return pl.pallas_call(_router_body, out_shape=out_shapes)(x2, wr_p, br_row)


def _dispatch_body(x_hbm, pos0_hbm, pos1_hbm, w0_hbm, w1_hbm, xs_hbm, wgt_hbm,
                   rows_v, idx0_v, idx1_v, wv0_v, wv1_v, sem):
    wid = lax.axis_index("s") * NC + lax.axis_index("c")
    base = wid * TPW
    for h in range(TPW // CH):
        b = base + h * CH
        pltpu.sync_copy(x_hbm.at[pl.ds(b, CH)], rows_v)
        pltpu.sync_copy(pos0_hbm.at[pl.ds(b, CH)], idx0_v)
        pltpu.sync_copy(pos1_hbm.at[pl.ds(b, CH)], idx1_v)
        pltpu.sync_copy(w0_hbm.at[pl.ds(b, CH)], wv0_v)
        pltpu.sync_copy(w1_hbm.at[pl.ds(b, CH)], wv1_v)
        cp0 = pltpu.async_copy(rows_v, xs_hbm.at[idx0_v], sem)
        cp1 = pltpu.async_copy(rows_v, xs_hbm.at[idx1_v], sem)
        cp2 = pltpu.async_copy(wv0_v, wgt_hbm.at[idx0_v], sem)
        cp3 = pltpu.async_copy(wv1_v, wgt_hbm.at[idx1_v], sem)
        cp0.wait()
        cp1.wait()
        cp2.wait()
        cp3.wait()


def _dispatch(x2, p0, p1, w0, w1):
    mesh = plsc.VectorSubcoreMesh(core_axis_name="c", subcore_axis_name="s")
    fn = functools.partial(
        pl.kernel, mesh=mesh,
        out_type=(jax.ShapeDtypeStruct((NPAD, H), jnp.float32),
                  jax.ShapeDtypeStruct((NPAD,), jnp.float32)),
        scratch_types=[
            pltpu.VMEM((CH, H), jnp.float32),
            pltpu.VMEM((CH,), jnp.int32),
            pltpu.VMEM((CH,), jnp.int32),
            pltpu.VMEM((CH,), jnp.float32),
            pltpu.VMEM((CH,), jnp.float32),
            pltpu.SemaphoreType.DMA,
        ],
    )(_dispatch_body)
    return fn(x2, p0, p1, w0, w1)


CHC = 32


def _combine_body(ys_hbm, pos0_hbm, pos1_hbm, out_hbm,
                  r0_v, r1_v, idx0_v, idx1_v, sem):
    wid = lax.axis_index("s") * NC + lax.axis_index("c")
    base = wid * TPW
    for h in range(TPW // CHC):
        b = base + h * CHC
        pltpu.sync_copy(pos0_hbm.at[pl.ds(b, CHC)], idx0_v)
        pltpu.sync_copy(pos1_hbm.at[pl.ds(b, CHC)], idx1_v)
        cp0 = pltpu.async_copy(ys_hbm.at[idx0_v], r0_v, sem)
        cp1 = pltpu.async_copy(ys_hbm.at[idx1_v], r1_v, sem)
        cp0.wait()
        cp1.wait()
        for r in range(CHC):
            def _add(c, _):
                sl = pl.ds(c * 16, 16)
                r0_v[r, sl] = r0_v[r, sl] + r1_v[r, sl]
                return ()
            lax.fori_loop(0, H // 16, _add, (), unroll=4)
        pltpu.sync_copy(r0_v, out_hbm.at[pl.ds(b, CHC)])


def _combine(ys, p0, p1):
    mesh = plsc.VectorSubcoreMesh(core_axis_name="c", subcore_axis_name="s")
    fn = functools.partial(
        pl.kernel, mesh=mesh,
        out_type=jax.ShapeDtypeStruct((S, H), jnp.float32),
        scratch_types=[
            pltpu.VMEM((CHC, H), jnp.float32),
            pltpu.VMEM((CHC, H), jnp.float32),
            pltpu.VMEM((CHC,), jnp.int32),
            pltpu.VMEM((CHC,), jnp.int32),
            pltpu.SemaphoreType.DMA,
        ],
    )(_combine_body)
    return fn(ys, p0, p1)


def _ffn_body(te_ref, tv_ref, xs_ref, wg_ref, w1_ref, b1_ref, w2_ref, b2_ref,
              ys_ref):
    t = pl.program_id(0)

    @pl.when(tv_ref[t] > 0)
    def _():
        z = jnp.dot(xs_ref[...], w1_ref[0], preferred_element_type=jnp.float32)
        z = z + b1_ref[0]
        h = jnp.maximum(z, 0.0)
        h = h * h
        y = jnp.dot(h, w2_ref[0], preferred_element_type=jnp.float32)
        ys_ref[...] = (y + b2_ref[0]) * wg_ref[...]


def _ffn(te, tv, xs, wgt, W1, b1, W2, b2):
    grid_spec = pltpu.PrefetchScalarGridSpec(
        num_scalar_prefetch=2,
        grid=(NT,),
        in_specs=[
            pl.BlockSpec((T, H), lambda t, te, tv: (t, 0)),
            pl.BlockSpec((T, 1), lambda t, te, tv: (t, 0)),
            pl.BlockSpec((1, H, DFF), lambda t, te, tv: (te[t], 0, 0)),
            pl.BlockSpec((1, 1, DFF), lambda t, te, tv: (te[t], 0, 0)),
            pl.BlockSpec((1, DFF, H), lambda t, te, tv: (te[t], 0, 0)),
            pl.BlockSpec((1, 1, H), lambda t, te, tv: (te[t], 0, 0)),
        ],
        out_specs=pl.BlockSpec((T, H), lambda t, te, tv: (t, 0)),
    )
    return pl.pallas_call(
        _ffn_body, grid_spec=grid_spec,
        out_shape=jax.ShapeDtypeStruct((NPAD, H), jnp.float32),
        compiler_params=pltpu.CompilerParams(
            dimension_semantics=("arbitrary",)),
    )(te, tv, xs, wgt, W1.astype(jnp.bfloat16), b1.reshape(E, 1, DFF), W2,
      b2.reshape(E, 1, H))


def kernel(x, Wr, br, W1, b1, W2, b2):
    x2 = x.reshape(S, H)
    wr_p = jnp.pad(Wr, ((0, 0), (0, EPAD - E)))
    pos0, pos1, w0, w1, te_c, tv_c, bal = _router(x2, wr_p, br.reshape(1, E))
    te = te_c.reshape(-1)[:NT]
    tv = tv_c.reshape(-1)[:NT]
    p0 = pos0.reshape(-1)
    p1 = pos1.reshape(-1)
    xs, wgt = _dispatch(x2, p0, p1, w0.reshape(-1), w1.reshape(-1))
    ys = _ffn(te, tv, xs, wgt.reshape(NPAD, 1), W1, b1, W2, b2)
    out = _combine(ys, p0, p1)
    return out.reshape(1, S, H), bal[0, 0]


# speedup vs baseline: 1.1313x; 1.1313x over previous
import functools

import jax
import jax.numpy as jnp
from jax import lax
from jax.experimental import pallas as pl
from jax.experimental.pallas import tpu as pltpu
from jax.experimental.pallas import tpu_sc as plsc

S, H, E, DFF = 4096, 1024, 8, 4096
T = 256
NF = 4
FB = DFF // NF
NT = 40
NPAD = NT * T
EPAD = 128

NC, NS = 2, 16
NW = NC * NS
TPW = S // NW
CH = 64


def _shift_down(a, k):
    z = jnp.zeros((k,) + a.shape[1:], a.dtype)
    return jnp.concatenate([z, a[:-k]], axis=0)


def _router_body(x_ref, wr_ref, br_ref,
                 pos0_ref, pos1_ref, w0_ref, w1_ref, te_ref, tv_ref, bal_ref):
    xv = x_ref[...]
    scores_p = jnp.dot(xv, wr_ref[...], preferred_element_type=jnp.float32)
    scores = scores_p[:, 0:E] + br_ref[...]
    m = jnp.max(scores, axis=1, keepdims=True)
    ex = jnp.exp(scores - m)
    probs = ex / jnp.sum(ex, axis=1, keepdims=True)

    iota_e = jax.lax.broadcasted_iota(jnp.int32, (S, E), 1)
    p0 = jnp.max(probs, axis=1, keepdims=True)
    i0 = jnp.min(jnp.where(probs == p0, iota_e, E), axis=1, keepdims=True)
    oh0 = (iota_e == i0).astype(jnp.int32)
    probs2 = jnp.where(oh0 == 1, -jnp.inf, probs)
    p1 = jnp.max(probs2, axis=1, keepdims=True)
    i1 = jnp.min(jnp.where(probs2 == p1, iota_e, E), axis=1, keepdims=True)
    oh1 = (iota_e == i1).astype(jnp.int32)

    psum = p0 + p1
    w0_ref[...] = p0 / psum
    w1_ref[...] = p1 / psum

    cb = jnp.concatenate([oh0, oh1], axis=1)
    k = 1
    while k < S:
        cb = cb + _shift_down(cb, k)
        k *= 2
    c0 = cb[:, 0:E]
    c1 = cb[:, E:2 * E]
    total0 = c0[S - 1:S, :]
    total1 = c1[S - 1:S, :]
    counts = total0 + total1

    cf = counts.astype(jnp.float32)
    ntT = jnp.ceil(cf / T) * T
    cum = ntT
    for kk in (1, 2, 4):
        z = jnp.zeros((1, kk), cum.dtype)
        cum = cum + jnp.concatenate([z, cum[:, :-kk]], axis=1)
    pstart = (cum - ntT).astype(jnp.int32)

    rank0 = jnp.sum((c0 - 1) * oh0, axis=1, keepdims=True)
    rank1 = jnp.sum((c1 - 1) * oh1 + total0 * oh1, axis=1, keepdims=True)
    pos0_ref[...] = rank0 + jnp.sum(pstart * oh0, axis=1, keepdims=True)
    pos1_ref[...] = rank1 + jnp.sum(pstart * oh1, axis=1, keepdims=True)

    cum_t = (cum * (1.0 / T)).astype(jnp.int32)
    t_iota = jax.lax.broadcasted_iota(jnp.int32, (128, E), 0)
    te = jnp.sum((t_iota >= cum_t).astype(jnp.int32), axis=1, keepdims=True)
    te_ref[...] = jnp.minimum(te, E - 1)
    tv_ref[...] = (t_iota[:, 0:1] < cum_t[:, E - 1:E]).astype(jnp.int32)

    avgp = jnp.sum(probs, axis=0, keepdims=True) * (1.0 / S)
    freqs = total0.astype(jnp.float32) * (1.0 / S)
    bal_ref[...] = 0.001 * jnp.sum(freqs * avgp, axis=1, keepdims=True)


def _router(x2, wr_p, br_row):
    out_shapes = (
        jax.ShapeDtypeStruct((S, 1), jnp.int32),
        jax.ShapeDtypeStruct((S, 1), jnp.int32),
        jax.ShapeDtypeStruct((S, 1), jnp.float32),
        jax.ShapeDtypeStruct((S, 1), jnp.float32),
        jax.ShapeDtypeStruct((128, 1), jnp.int32),
        jax.ShapeDtypeStruct((128, 1), jnp.int32),
        jax.ShapeDtypeStruct((1, 1), jnp.float32),
    )
    return pl.pallas_call(_router_body, out_shape=out_shapes)(x2, wr_p, br_row)


def _dispatch_body(x_hbm, pos0_hbm, pos1_hbm, xs_hbm,
                   rows_v, idx0_v, idx1_v, sem):
    wid = lax.axis_index("s") * NC + lax.axis_index("c")
    base = wid * TPW
    for h in range(TPW // CH):
        b = base + h * CH
        pltpu.sync_copy(x_hbm.at[pl.ds(b, CH)], rows_v)
        pltpu.sync_copy(pos0_hbm.at[pl.ds(b, CH)], idx0_v)
        pltpu.sync_copy(pos1_hbm.at[pl.ds(b, CH)], idx1_v)
        cp0 = pltpu.async_copy(rows_v, xs_hbm.at[idx0_v], sem)
        cp1 = pltpu.async_copy(rows_v, xs_hbm.at[idx1_v], sem)
        cp0.wait()
        cp1.wait()


def _dispatch(x2, p0, p1):
    mesh = plsc.VectorSubcoreMesh(core_axis_name="c", subcore_axis_name="s")
    fn = functools.partial(
        pl.kernel, mesh=mesh,
        out_type=jax.ShapeDtypeStruct((NPAD, H), jnp.float32),
        scratch_types=[
            pltpu.VMEM((CH, H), jnp.float32),
            pltpu.VMEM((CH,), jnp.int32),
            pltpu.VMEM((CH,), jnp.int32),
            pltpu.SemaphoreType.DMA,
        ],
    )(_dispatch_body)
    return fn(x2, p0, p1)


def _combine_body(ys_hbm, pos0_hbm, pos1_hbm, g0_hbm, g1_hbm,
                  rows_v, idx_v, sem):
    wid = lax.axis_index("s") * NC + lax.axis_index("c")
    base = wid * TPW
    for h in range(TPW // CH):
        b = base + h * CH
        pltpu.sync_copy(pos0_hbm.at[pl.ds(b, CH)], idx_v)
        pltpu.async_copy(ys_hbm.at[idx_v], rows_v, sem).wait()
        pltpu.sync_copy(rows_v, g0_hbm.at[pl.ds(b, CH)])
        pltpu.sync_copy(pos1_hbm.at[pl.ds(b, CH)], idx_v)
        pltpu.async_copy(ys_hbm.at[idx_v], rows_v, sem).wait()
        pltpu.sync_copy(rows_v, g1_hbm.at[pl.ds(b, CH)])


def _combine(ys, p0, p1):
    mesh = plsc.VectorSubcoreMesh(core_axis_name="c", subcore_axis_name="s")
    fn = functools.partial(
        pl.kernel, mesh=mesh,
        out_type=(jax.ShapeDtypeStruct((S, H), jnp.float32),
                  jax.ShapeDtypeStruct((S, H), jnp.float32)),
        scratch_types=[
            pltpu.VMEM((CH, H), jnp.float32),
            pltpu.VMEM((CH,), jnp.int32),
            pltpu.SemaphoreType.DMA,
        ],
    )(_combine_body)
    return fn(ys, p0, p1)


def _ffn_body(te_ref, tv_ref, xs_ref, w1_ref, b1_ref, w2_ref, b2_ref, ys_ref):
    t = pl.program_id(0)

    @pl.when(tv_ref[t] > 0)
    def _():
        z = jnp.dot(xs_ref[...], w1_ref[0], preferred_element_type=jnp.float32)
        z = z + b1_ref[0]
        h = jnp.maximum(z, 0.0)
        h = h * h
        y = jnp.dot(h, w2_ref[0], preferred_element_type=jnp.float32)
        ys_ref[...] = y + b2_ref[0]


def _ffn(te, tv, xs, W1, b1, W2, b2):
    grid_spec = pltpu.PrefetchScalarGridSpec(
        num_scalar_prefetch=2,
        grid=(NT,),
        in_specs=[
            pl.BlockSpec((T, H), lambda t, te, tv: (t, 0)),
            pl.BlockSpec((1, H, DFF), lambda t, te, tv: (te[t], 0, 0)),
            pl.BlockSpec((1, 1, DFF), lambda t, te, tv: (te[t], 0, 0)),
            pl.BlockSpec((1, DFF, H), lambda t, te, tv: (te[t], 0, 0)),
            pl.BlockSpec((1, 1, H), lambda t, te, tv: (te[t], 0, 0)),
        ],
        out_specs=pl.BlockSpec((T, H), lambda t, te, tv: (t, 0)),
    )
    return pl.pallas_call(
        _ffn_body, grid_spec=grid_spec,
        out_shape=jax.ShapeDtypeStruct((NPAD, H), jnp.float32),
        compiler_params=pltpu.CompilerParams(
            dimension_semantics=("arbitrary",)),
    )(te, tv, xs, W1.astype(jnp.bfloat16), b1.reshape(E, 1, DFF), W2,
      b2.reshape(E, 1, H))


def _finalize_body(a_ref, b_ref, w0_ref, w1_ref, o_ref):
    o_ref[...] = w0_ref[...] * a_ref[...] + w1_ref[...] * b_ref[...]


def _finalize(g0, g1, w0, w1):
    TS = 512
    return pl.pallas_call(
        _finalize_body,
        grid=(S // TS,),
        in_specs=[
            pl.BlockSpec((TS, H), lambda t: (t, 0)),
            pl.BlockSpec((TS, H), lambda t: (t, 0)),
            pl.BlockSpec((TS, 1), lambda t: (t, 0)),
            pl.BlockSpec((TS, 1), lambda t: (t, 0)),
        ],
        out_specs=pl.BlockSpec((TS, H), lambda t: (t, 0)),
        out_shape=jax.ShapeDtypeStruct((S, H), jnp.float32),
    )(g0, g1, w0, w1)


def kernel(x, Wr, br, W1, b1, W2, b2):
    x2 = x.reshape(S, H)
    wr_p = jnp.pad(Wr, ((0, 0), (0, EPAD - E)))
    pos0, pos1, w0, w1, te_c, tv_c, bal = _router(x2, wr_p, br.reshape(1, E))
    te = te_c.reshape(-1)[:NT]
    tv = tv_c.reshape(-1)[:NT]
    p0 = pos0.reshape(-1)
    p1 = pos1.reshape(-1)
    xs = _dispatch(x2, p0, p1)
    ys = _ffn(te, tv, xs, W1, b1, W2, b2)
    g0, g1 = _combine(ys, p0, p1)
    out = _finalize(g0, g1, w0, w1)
    return out.reshape(1, S, H), bal[0, 0]


# speedup vs baseline: 1.1344x; 1.0027x over previous
import functools

import jax
import jax.numpy as jnp
from jax import lax
from jax.experimental import pallas as pl
from jax.experimental.pallas import tpu as pltpu
from jax.experimental.pallas import tpu_sc as plsc

S, H, E, DFF = 4096, 1024, 8, 4096
T = 256
NF = 4
FB = DFF // NF
NT = 40
NPAD = NT * T
EPAD = 128

NC, NS = 2, 16
NW = NC * NS
TPW = S // NW
CH = 64


def _shift_down(a, k):
    z = jnp.zeros((k,) + a.shape[1:], a.dtype)
    return jnp.concatenate([z, a[:-k]], axis=0)


def _router_body(x_ref, wr_ref, br_ref,
                 pos0_ref, pos1_ref, w0_ref, w1_ref, te_ref, tv_ref, bal_ref):
    xv = x_ref[...]
    scores_p = jnp.dot(xv, wr_ref[...], preferred_element_type=jnp.float32)
    scores = scores_p[:, 0:E] + br_ref[...]
    m = jnp.max(scores, axis=1, keepdims=True)
    ex = jnp.exp(scores - m)
    probs = ex / jnp.sum(ex, axis=1, keepdims=True)

    iota_e = jax.lax.broadcasted_iota(jnp.int32, (S, E), 1)
    p0 = jnp.max(probs, axis=1, keepdims=True)
    i0 = jnp.min(jnp.where(probs == p0, iota_e, E), axis=1, keepdims=True)
    oh0 = (iota_e == i0).astype(jnp.int32)
    probs2 = jnp.where(oh0 == 1, -jnp.inf, probs)
    p1 = jnp.max(probs2, axis=1, keepdims=True)
    i1 = jnp.min(jnp.where(probs2 == p1, iota_e, E), axis=1, keepdims=True)
    oh1 = (iota_e == i1).astype(jnp.int32)

    psum = p0 + p1
    w0_ref[...] = p0 / psum
    w1_ref[...] = p1 / psum

    cb = jnp.concatenate([oh0, oh1], axis=1)
    k = 1
    while k < S:
        cb = cb + _shift_down(cb, k)
        k *= 2
    c0 = cb[:, 0:E]
    c1 = cb[:, E:2 * E]
    total0 = c0[S - 1:S, :]
    total1 = c1[S - 1:S, :]
    counts = total0 + total1

    cf = counts.astype(jnp.float32)
    ntT = jnp.ceil(cf / T) * T
    cum = ntT
    for kk in (1, 2, 4):
        z = jnp.zeros((1, kk), cum.dtype)
        cum = cum + jnp.concatenate([z, cum[:, :-kk]], axis=1)
    pstart = (cum - ntT).astype(jnp.int32)

    rank0 = jnp.sum((c0 - 1) * oh0, axis=1, keepdims=True)
    rank1 = jnp.sum((c1 - 1) * oh1 + total0 * oh1, axis=1, keepdims=True)
    pos0_ref[...] = rank0 + jnp.sum(pstart * oh0, axis=1, keepdims=True)
    pos1_ref[...] = rank1 + jnp.sum(pstart * oh1, axis=1, keepdims=True)

    cum_t = (cum * (1.0 / T)).astype(jnp.int32)
    t_iota = jax.lax.broadcasted_iota(jnp.int32, (128, E), 0)
    te = jnp.sum((t_iota >= cum_t).astype(jnp.int32), axis=1, keepdims=True)
    te_ref[...] = jnp.minimum(te, E - 1)
    tv_ref[...] = (t_iota[:, 0:1] < cum_t[:, E - 1:E]).astype(jnp.int32)

    avgp = jnp.sum(probs, axis=0, keepdims=True) * (1.0 / S)
    freqs = total0.astype(jnp.float32) * (1.0 / S)
    bal_ref[...] = 0.001 * jnp.sum(freqs * avgp, axis=1, keepdims=True)


def _router(x2, wr_p, br_row):
    out_shapes = (
        jax.ShapeDtypeStruct((S, 1), jnp.int32),
        jax.ShapeDtypeStruct((S, 1), jnp.int32),
        jax.ShapeDtypeStruct((S, 1), jnp.float32),
        jax.ShapeDtypeStruct((S, 1), jnp.float32),
        jax.ShapeDtypeStruct((128, 1), jnp.int32),
        jax.ShapeDtypeStruct((128, 1), jnp.int32),
        jax.ShapeDtypeStruct((1, 1), jnp.float32),
    )
    return pl.pallas_call(_router_body, out_shape=out_shapes)(x2, wr_p, br_row)


CHD = 32


def _dispatch_body(x_hbm, pos0_hbm, pos1_hbm, xs_hbm,
                   ra_v, rb_v, i0a_v, i0b_v, i1a_v, i1b_v, sem):
    wid = lax.axis_index("s") * NC + lax.axis_index("c")
    base = wid * TPW
    nch = TPW // CHD
    rbuf = (ra_v, rb_v)
    ibuf0 = (i0a_v, i0b_v)
    ibuf1 = (i1a_v, i1b_v)
    pltpu.sync_copy(x_hbm.at[pl.ds(base, CHD)], ra_v)
    pltpu.sync_copy(pos0_hbm.at[pl.ds(base, CHD)], i0a_v)
    pltpu.sync_copy(pos1_hbm.at[pl.ds(base, CHD)], i1a_v)
    for h in range(nch):
        cur = h % 2
        nxt = (h + 1) % 2
        cp0 = pltpu.async_copy(rbuf[cur], xs_hbm.at[ibuf0[cur]], sem)
        cp1 = pltpu.async_copy(rbuf[cur], xs_hbm.at[ibuf1[cur]], sem)
        if h + 1 < nch:
            b2 = base + (h + 1) * CHD
            pltpu.sync_copy(x_hbm.at[pl.ds(b2, CHD)], rbuf[nxt])
            pltpu.sync_copy(pos0_hbm.at[pl.ds(b2, CHD)], ibuf0[nxt])
            pltpu.sync_copy(pos1_hbm.at[pl.ds(b2, CHD)], ibuf1[nxt])
        cp0.wait()
        cp1.wait()


def _dispatch(x2, p0, p1):
    mesh = plsc.VectorSubcoreMesh(core_axis_name="c", subcore_axis_name="s")
    fn = functools.partial(
        pl.kernel, mesh=mesh,
        out_type=jax.ShapeDtypeStruct((NPAD, H), jnp.float32),
        scratch_types=[
            pltpu.VMEM((CHD, H), jnp.float32),
            pltpu.VMEM((CHD, H), jnp.float32),
            pltpu.VMEM((CHD,), jnp.int32),
            pltpu.VMEM((CHD,), jnp.int32),
            pltpu.VMEM((CHD,), jnp.int32),
            pltpu.VMEM((CHD,), jnp.int32),
            pltpu.SemaphoreType.DMA,
        ],
    )(_dispatch_body)
    return fn(x2, p0, p1)


def _combine_body(ys_hbm, pos0_hbm, pos1_hbm, g0_hbm, g1_hbm,
                  ra_v, rb_v, ia_v, ib_v, sema, semb):
    wid = lax.axis_index("s") * NC + lax.axis_index("c")
    base = wid * TPW
    nun = 2 * (TPW // CHD)
    rbuf = (ra_v, rb_v)
    ibuf = (ia_v, ib_v)
    sems = (sema, semb)
    pos = (pos0_hbm, pos1_hbm)
    dst = (g0_hbm, g1_hbm)
    pltpu.sync_copy(pos0_hbm.at[pl.ds(base, CHD)], ia_v)
    cps = [None, None]
    cps[0] = pltpu.async_copy(ys_hbm.at[ia_v], ra_v, sema)
    for u in range(nun):
        cur = u % 2
        nxt = (u + 1) % 2
        if u + 1 < nun:
            v = u + 1
            bv = base + (v // 2) * CHD
            pltpu.sync_copy(pos[v % 2].at[pl.ds(bv, CHD)], ibuf[nxt])
            cps[nxt] = pltpu.async_copy(ys_hbm.at[ibuf[nxt]], rbuf[nxt],
                                        sems[nxt])
        cps[cur].wait()
        bu = base + (u // 2) * CHD
        pltpu.sync_copy(rbuf[cur], dst[u % 2].at[pl.ds(bu, CHD)])


def _combine(ys, p0, p1):
    mesh = plsc.VectorSubcoreMesh(core_axis_name="c", subcore_axis_name="s")
    fn = functools.partial(
        pl.kernel, mesh=mesh,
        out_type=(jax.ShapeDtypeStruct((S, H), jnp.float32),
                  jax.ShapeDtypeStruct((S, H), jnp.float32)),
        scratch_types=[
            pltpu.VMEM((CHD, H), jnp.float32),
            pltpu.VMEM((CHD, H), jnp.float32),
            pltpu.VMEM((CHD,), jnp.int32),
            pltpu.VMEM((CHD,), jnp.int32),
            pltpu.SemaphoreType.DMA,
            pltpu.SemaphoreType.DMA,
        ],
    )(_combine_body)
    return fn(ys, p0, p1)


def _ffn_body(te_ref, tv_ref, xs_ref, w1_ref, b1_ref, w2_ref, b2_ref, ys_ref):
    t = pl.program_id(0)

    @pl.when(tv_ref[t] > 0)
    def _():
        z = jnp.dot(xs_ref[...], w1_ref[0], preferred_element_type=jnp.float32)
        z = z + b1_ref[0]
        h = jnp.maximum(z, 0.0)
        h = h * h
        y = jnp.dot(h, w2_ref[0], preferred_element_type=jnp.float32)
        ys_ref[...] = y + b2_ref[0]


def _ffn(te, tv, xs, W1, b1, W2, b2):
    grid_spec = pltpu.PrefetchScalarGridSpec(
        num_scalar_prefetch=2,
        grid=(NT,),
        in_specs=[
            pl.BlockSpec((T, H), lambda t, te, tv: (t, 0)),
            pl.BlockSpec((1, H, DFF), lambda t, te, tv: (te[t], 0, 0)),
            pl.BlockSpec((1, 1, DFF), lambda t, te, tv: (te[t], 0, 0)),
            pl.BlockSpec((1, DFF, H), lambda t, te, tv: (te[t], 0, 0)),
            pl.BlockSpec((1, 1, H), lambda t, te, tv: (te[t], 0, 0)),
        ],
        out_specs=pl.BlockSpec((T, H), lambda t, te, tv: (t, 0)),
    )
    return pl.pallas_call(
        _ffn_body, grid_spec=grid_spec,
        out_shape=jax.ShapeDtypeStruct((NPAD, H), jnp.float32),
        compiler_params=pltpu.CompilerParams(
            dimension_semantics=("arbitrary",)),
    )(te, tv, xs, W1.astype(jnp.bfloat16), b1.reshape(E, 1, DFF), W2,
      b2.reshape(E, 1, H))


def _finalize_body(a_ref, b_ref, w0_ref, w1_ref, o_ref):
    o_ref[...] = (w0_ref[...] * a_ref[...].astype(jnp.float32)
                  + w1_ref[...] * b_ref[...].astype(jnp.float32))


def _finalize(g0, g1, w0, w1):
    TS = 512
    return pl.pallas_call(
        _finalize_body,
        grid=(S // TS,),
        in_specs=[
            pl.BlockSpec((TS, H), lambda t: (t, 0)),
            pl.BlockSpec((TS, H), lambda t: (t, 0)),
            pl.BlockSpec((TS, 1), lambda t: (t, 0)),
            pl.BlockSpec((TS, 1), lambda t: (t, 0)),
        ],
        out_specs=pl.BlockSpec((TS, H), lambda t: (t, 0)),
        out_shape=jax.ShapeDtypeStruct((S, H), jnp.float32),
    )(g0, g1, w0, w1)


def kernel(x, Wr, br, W1, b1, W2, b2):
    x2 = x.reshape(S, H)
    wr_p = jnp.pad(Wr, ((0, 0), (0, EPAD - E)))
    pos0, pos1, w0, w1, te_c, tv_c, bal = _router(x2, wr_p, br.reshape(1, E))
    te = te_c.reshape(-1)[:NT]
    tv = tv_c.reshape(-1)[:NT]
    p0 = pos0.reshape(-1)
    p1 = pos1.reshape(-1)
    xs = _dispatch(x2, p0, p1)
    ys = _ffn(te, tv, xs, W1, b1, W2, b2)
    g0, g1 = _combine(ys, p0, p1)
    out = _finalize(g0, g1, w0, w1)
    return out.reshape(1, S, H), bal[0, 0]
